# baseline (speedup 1.0000x reference)
import functools
import math

import jax
import jax.numpy as jnp
from jax import lax
from jax.experimental import pallas as pl
from jax.experimental.pallas import tpu as pltpu
from jax.experimental.pallas import tpu_sc as plsc

D_MODEL = 64
SCALE = math.sqrt(D_MODEL)

NUM_CORES = 2
NUM_SUBCORES = 16
NUM_WORKERS = NUM_CORES * NUM_SUBCORES
LANES = 16

NBUF = 4


def _emb_kernel(idx_hbm, table_hbm, out_hbm, idx_v, rows, gsems, osems):
    wid = lax.axis_index("s") * NUM_CORES + lax.axis_index("c")
    rows_per_w = idx_hbm.shape[0] // NUM_WORKERS
    seq = idx_hbm.shape[1]
    base = wid * rows_per_w

    def gather(r, b):
        pltpu.async_copy(table_hbm.at[idx_v.at[r]], rows[b], gsems[b])

    def gather_wait(b):
        pltpu.make_async_copy(table_hbm.at[idx_v.at[0]], rows[b], gsems[b]).wait()

    def writeback(r, b):
        pltpu.async_copy(rows[b], out_hbm.at[base + r], osems[b])

    def writeback_wait(b):
        pltpu.make_async_copy(rows[b], out_hbm.at[base], osems[b]).wait()

    pltpu.sync_copy(idx_hbm.at[pl.ds(base, rows_per_w)], idx_v)
    gather(0, 0)

    def group(g, _):
        for b in range(NBUF):
            r = g * NBUF + b
            b_next = (b + 1) % NBUF

            @pl.when(r >= NBUF - 1)
            def _():
                writeback_wait(b_next)

            @pl.when(r + 1 < rows_per_w)
            def _():
                gather(r + 1, b_next)

            gather_wait(b)

            @plsc.parallel_loop(0, seq, step=1, unroll=8)
            def _(t):
                for j in range(D_MODEL // LANES):
                    sl = pl.ds(j * LANES, LANES)
                    rows[b][t, sl] = rows[b][t, sl] * SCALE

            writeback(r, b)
        return 0

    lax.fori_loop(0, rows_per_w // NBUF, group, 0)

    for k in range(1, NBUF):
        writeback_wait((rows_per_w - k) % NBUF)


def kernel(x, table):
    batch, seq = x.shape
    rows_per_w = batch // NUM_WORKERS

    mesh = plsc.VectorSubcoreMesh(core_axis_name="c", subcore_axis_name="s")
    run = pl.kernel(
        _emb_kernel,
        out_type=jax.ShapeDtypeStruct((batch, seq, D_MODEL), jnp.float32),
        mesh=mesh,
        scratch_types=[
            pltpu.VMEM((rows_per_w, seq), jnp.int32),
            [pltpu.VMEM((seq, D_MODEL), jnp.float32) for _ in range(NBUF)],
            [pltpu.SemaphoreType.DMA for _ in range(NBUF)],
            [pltpu.SemaphoreType.DMA for _ in range(NBUF)],
        ],
        compiler_params=pltpu.CompilerParams(use_tc_tiling_on_sc=False),
    )
    return run(x, table)


# speedup vs baseline: 1.0002x; 1.0002x over previous
import functools
import math

import jax
import jax.numpy as jnp
from jax import lax
from jax.experimental import pallas as pl
from jax.experimental.pallas import tpu as pltpu
from jax.experimental.pallas import tpu_sc as plsc

D_MODEL = 64
SCALE = math.sqrt(D_MODEL)

NUM_CORES = 2
NUM_SUBCORES = 16
NUM_WORKERS = NUM_CORES * NUM_SUBCORES
LANES = 16

NBUF = 4


def _emb_kernel(idx_hbm, table_hbm, out_hbm, idx_v, rows, gsems, osems):
    wid = lax.axis_index("s") * NUM_CORES + lax.axis_index("c")
    rows_per_w = idx_hbm.shape[0] // NUM_WORKERS
    seq = idx_hbm.shape[1]
    base = wid * rows_per_w

    def gather(r, b):
        pltpu.async_copy(table_hbm.at[idx_v.at[r]], rows[b], gsems[b])

    def gather_wait(b):
        pltpu.make_async_copy(table_hbm.at[idx_v.at[0]], rows[b], gsems[b]).wait()

    def writeback(r, b):
        pltpu.async_copy(rows[b], out_hbm.at[base + r], osems[b])

    def writeback_wait(b):
        pltpu.make_async_copy(rows[b], out_hbm.at[base], osems[b]).wait()

    pltpu.sync_copy(idx_hbm.at[pl.ds(base, rows_per_w)], idx_v)
    gather(0, 0)

    def group(g, _):
        for b in range(NBUF):
            r = g * NBUF + b
            b_next = (b + 1) % NBUF

            @pl.when(r >= NBUF - 1)
            def _():
                writeback_wait(b_next)

            @pl.when(r + 1 < rows_per_w)
            def _():
                gather(r + 1, b_next)

            gather_wait(b)

            @plsc.parallel_loop(0, seq, step=1, unroll=8)
            def _(t):
                for j in range(D_MODEL // LANES):
                    sl = pl.ds(j * LANES, LANES)
                    rows[b][t, sl] = rows[b][t, sl] * SCALE

            writeback(r, b)
        return 0

    lax.fori_loop(0, rows_per_w // NBUF, group, 0)

    for k in range(1, NBUF):
        writeback_wait((rows_per_w - k) % NBUF)


def kernel(x, table):
    batch, seq = x.shape
    rows_per_w = batch // NUM_WORKERS

    mesh = plsc.VectorSubcoreMesh(core_axis_name="c", subcore_axis_name="s")
    run = pl.kernel(
        _emb_kernel,
        out_type=jax.ShapeDtypeStruct((batch, seq, D_MODEL), jnp.float32),
        mesh=mesh,
        scratch_types=[
            pltpu.VMEM((rows_per_w, seq), jnp.int32),
            [pltpu.VMEM((seq, D_MODEL), jnp.float32) for _ in range(NBUF)],
            [pltpu.SemaphoreType.DMA for _ in range(NBUF)],
            [pltpu.SemaphoreType.DMA for _ in range(NBUF)],
        ],
        compiler_params=pltpu.CompilerParams(
            use_tc_tiling_on_sc=False, skip_device_barrier=True
        ),
    )
    return run(x, table)
